# baseline (speedup 1.0000x reference)
import functools

import jax
import jax.numpy as jnp
from jax import lax
from jax.experimental import pallas as pl
from jax.experimental.pallas import tpu as pltpu
from jax.experimental.pallas import tpu_sc as plsc

DIM = 32
GRP = 128
TPAD = 131
NBUF = 4

_info = plsc.get_sparse_core_info()
_NC, _NS = _info.num_cores, _info.num_subcores
_NW = _NC * _NS


@functools.partial(jax.jit, static_argnums=(2, 3))
def _sc_gather(ids_lin, table_lin, seq, nb):
    n_groups = seq * nb
    per_w = n_groups // _NW
    mesh = plsc.VectorSubcoreMesh(core_axis_name="c", subcore_axis_name="s")

    @functools.partial(
        pl.kernel,
        out_type=jax.ShapeDtypeStruct((seq, DIM // 8, nb, 8, GRP), jnp.float32),
        mesh=mesh,
        scratch_types=(
            [pltpu.VMEM((per_w * GRP,), jnp.int32)]
            + [pltpu.VMEM((GRP, DIM), jnp.float32) for _ in range(NBUF)]
            + [pltpu.VMEM((DIM, TPAD), jnp.float32) for _ in range(NBUF)]
            + [pltpu.SemaphoreType.DMA for _ in range(2 * NBUF)]
        ),
        compiler_params=pltpu.CompilerParams(
            use_tc_tiling_on_sc=False, needs_layout_passes=False),
    )
    def k(ids_hbm, tab_hbm, out_hbm, idx_v, *rest):
        rows = rest[:NBUF]
        tiles = rest[NBUF:2 * NBUF]
        gsems = rest[2 * NBUF:3 * NBUF]
        ssems = rest[3 * NBUF:]
        wid = lax.axis_index("s") * _NC + lax.axis_index("c")
        gbase = wid * per_w
        pltpu.sync_copy(ids_hbm.at[pl.ds(gbase * GRP, per_w * GRP)], idx_v)

        iota16 = lax.broadcasted_iota(jnp.int32, (16,), 0)
        dvec = [iota16 + 16 * h for h in range(2)]
        zero16 = jnp.zeros((16,), jnp.int32)

        def gather(g, q):
            pltpu.async_copy(
                tab_hbm.at[idx_v.at[pl.ds(g * GRP, GRP)]], rows[q], gsems[q])

        def gather_wait(q):
            pltpu.make_async_copy(
                tab_hbm.at[pl.ds(0, GRP)], rows[q], gsems[q]).wait()

        def transpose(q, p):
            rv, tv = rows[q], tiles[p]
            for b in range(GRP):
                bidx = zero16 + b
                for h in range(2):
                    v = rv[b, pl.ds(16 * h, 16)]
                    plsc.store_scatter(tv, [dvec[h], bidx], v)

        def store(g, p):
            s = (gbase + g) // nb
            b = (gbase + g) % nb
            for j in range(DIM // 8):
                pltpu.async_copy(
                    tiles[p].at[pl.ds(8 * j, 8), pl.ds(0, GRP)],
                    out_hbm.at[s, j, b], ssems[p])

        def store_wait(p):
            for j in range(DIM // 8):
                pltpu.make_async_copy(
                    tiles[p].at[pl.ds(8 * j, 8), pl.ds(0, GRP)],
                    out_hbm.at[0, j, 0], ssems[p]).wait()

        for q in range(NBUF):
            gather(q, q)

        def body(i, carry):
            for u in range(NBUF):
                g = NBUF * i + u
                q = p = u
                gather_wait(q)

                @pl.when(g >= NBUF)
                def _():
                    store_wait(p)

                transpose(q, p)

                @pl.when(g + NBUF < per_w)
                def _():
                    gather(g + NBUF, q)

                store(g, p)
            return carry

        lax.fori_loop(0, per_w // NBUF, body, 0)
        for p in range(NBUF):
            store_wait(p)

    return k(ids_lin, table_lin)


def kernel(input_ids, table):
    bsz, seq = input_ids.shape
    nb = bsz // GRP
    ids_lin = (input_ids * 4).T.reshape(-1)
    padded = jnp.pad(table, ((0, 0), (0, GRP - DIM)))
    table_lin = padded.reshape(-1, DIM)
    arr = _sc_gather(ids_lin, table_lin, seq, nb)
    out = arr.transpose(2, 4, 0, 1, 3).reshape(bsz, seq, DIM)
    return out


# speedup vs baseline: 1.0829x; 1.0829x over previous
import functools

import jax
import jax.numpy as jnp
from jax import lax
from jax.experimental import pallas as pl
from jax.experimental.pallas import tpu as pltpu
from jax.experimental.pallas import tpu_sc as plsc

DIM = 32
GRP = 128
TPAD = 131
NBUF = 2

_info = plsc.get_sparse_core_info()
_NC, _NS = _info.num_cores, _info.num_subcores
_NW = _NC * _NS


@functools.partial(jax.jit, static_argnums=(2, 3))
def _sc_gather(ids_lin, table_lin, seq, nb):
    n_groups = seq * nb
    per_w = n_groups // _NW
    mesh = plsc.VectorSubcoreMesh(core_axis_name="c", subcore_axis_name="s")

    @functools.partial(
        pl.kernel,
        out_type=jax.ShapeDtypeStruct((seq, DIM // 8, nb, 8, GRP), jnp.float32),
        mesh=mesh,
        scratch_types=(
            [pltpu.VMEM((per_w * GRP,), jnp.int32)]
            + [pltpu.VMEM((GRP, DIM), jnp.float32) for _ in range(NBUF)]
            + [pltpu.VMEM((DIM, TPAD), jnp.float32) for _ in range(NBUF)]
            + [pltpu.SemaphoreType.DMA for _ in range(2 * NBUF)]
        ),
        compiler_params=pltpu.CompilerParams(
            use_tc_tiling_on_sc=False, needs_layout_passes=False),
    )
    def k(ids_hbm, tab_hbm, out_hbm, idx_v, *rest):
        rows = rest[:NBUF]
        tiles = rest[NBUF:2 * NBUF]
        gsems = rest[2 * NBUF:3 * NBUF]
        ssems = rest[3 * NBUF:]
        wid = lax.axis_index("s") * _NC + lax.axis_index("c")
        gbase = wid * per_w
        pltpu.sync_copy(ids_hbm.at[pl.ds(gbase * GRP, per_w * GRP)], idx_v)

        iota16 = lax.broadcasted_iota(jnp.int32, (16,), 0)
        dvec = [iota16 + 16 * h for h in range(2)]
        zero16 = jnp.zeros((16,), jnp.int32)

        def gather(g, q):
            pltpu.async_copy(
                tab_hbm.at[idx_v.at[pl.ds(g * GRP, GRP)]], rows[q], gsems[q])

        def gather_wait(q):
            pltpu.make_async_copy(
                tab_hbm.at[pl.ds(0, GRP)], rows[q], gsems[q]).wait()

        def transpose(q, p):
            rv, tv = rows[q], tiles[p]
            for b in range(GRP):
                bidx = zero16 + b
                for h in range(2):
                    v = rv[b, pl.ds(16 * h, 16)]
                    plsc.store_scatter(tv, [dvec[h], bidx], v)

        def store(g, p):
            s = (gbase + g) // nb
            b = (gbase + g) % nb
            for j in range(DIM // 8):
                pltpu.async_copy(
                    tiles[p].at[pl.ds(8 * j, 8), pl.ds(0, GRP)],
                    out_hbm.at[s, j, b], ssems[p])

        def store_wait(p):
            for j in range(DIM // 8):
                pltpu.make_async_copy(
                    tiles[p].at[pl.ds(8 * j, 8), pl.ds(0, GRP)],
                    out_hbm.at[0, j, 0], ssems[p]).wait()

        for q in range(NBUF):
            gather(q, q)

        def body(i, carry):
            for u in range(NBUF):
                g = NBUF * i + u
                q = p = u
                gather_wait(q)

                @pl.when(g >= NBUF)
                def _():
                    store_wait(p)

                transpose(q, p)

                @pl.when(g + NBUF < per_w)
                def _():
                    gather(g + NBUF, q)

                store(g, p)
            return carry

        lax.fori_loop(0, per_w // NBUF, body, 0)
        for p in range(NBUF):
            store_wait(p)

    return k(ids_lin, table_lin)


def kernel(input_ids, table):
    bsz, seq = input_ids.shape
    nb = bsz // GRP
    ids_lin = (input_ids * 4).T.reshape(-1)
    padded = jnp.pad(table, ((0, 0), (0, GRP - DIM)))
    table_lin = padded.reshape(-1, DIM)
    arr = _sc_gather(ids_lin, table_lin, seq, nb)
    out = arr.transpose(2, 4, 0, 1, 3).reshape(bsz, seq, DIM)
    return out
